# baseline (speedup 1.0000x reference)
import jax
import jax.numpy as jnp
from jax.experimental import pallas as pl

N, E, DIN, DOUT, DTIME, H, B = 10000, 320000, 128, 128, 256, 4, 64


def _proj_kernel(x_ref, w_ref, o_ref):
    o_ref[...] = jnp.dot(x_ref[...], w_ref[...], preferred_element_type=jnp.float32)


def kernel(x, edge_index, time_emb, batch, W_gat, att_src, att_dst, bias_gat,
           W_time, b_time, ln_gamma, ln_beta):
    n = x.shape[0]
    BN = 400
    xh = pl.pallas_call(
        _proj_kernel,
        grid=(n // BN,),
        in_specs=[pl.BlockSpec((BN, DIN), lambda i: (i, 0)),
                  pl.BlockSpec((DIN, H * DOUT), lambda i: (0, 0))],
        out_specs=pl.BlockSpec((BN, H * DOUT), lambda i: (i, 0)),
        out_shape=jax.ShapeDtypeStruct((n, H * DOUT), jnp.float32),
    )(x, W_gat).reshape(n, H, DOUT)

    loop = jnp.arange(n, dtype=edge_index.dtype)
    src = jnp.concatenate([edge_index[0], loop])
    dst = jnp.concatenate([edge_index[1], loop])
    a_src = (xh * att_src[None, :, :]).sum(-1)
    a_dst = (xh * att_dst[None, :, :]).sum(-1)
    alpha = a_src[src] + a_dst[dst]
    alpha = jax.nn.leaky_relu(alpha, 0.2)
    amax = jax.ops.segment_max(alpha, dst, num_segments=n)
    alpha = jnp.exp(alpha - amax[dst])
    denom = jax.ops.segment_sum(alpha, dst, num_segments=n)
    alpha = alpha / (denom[dst] + 1e-16)
    msg = xh[src] * alpha[:, :, None]
    out = jax.ops.segment_sum(msg, dst, num_segments=n)
    h = out.mean(axis=1) + bias_gat
    t = jax.nn.silu(time_emb) @ W_time + b_time
    h = h + t[batch]
    mu = h.mean(-1, keepdims=True)
    var = ((h - mu) ** 2).mean(-1, keepdims=True)
    h = (h - mu) / jnp.sqrt(var + 1e-5) * ln_gamma + ln_beta
    return jax.nn.silu(h)
